# baseline (speedup 1.0000x reference)
import functools

import jax
import jax.numpy as jnp
from jax import lax
from jax.experimental import pallas as pl
from jax.experimental.pallas import tpu as pltpu
from jax.experimental.pallas import tpu_sc as plsc

_EMPHASIS = 1.5
_TOPK_PCT = 0.2

_SC_ROWS = 1024
_CHUNK = 16


def _sc_scores_kernel_factory(B, T, D):
    info = plsc.get_sparse_core_info()
    NW = info.num_cores * info.num_subcores
    rows_per_tile = B * _SC_ROWS // NW
    n_chunks = rows_per_tile // _CHUNK
    tiles_per_batch = NW // B
    t0 = T - _SC_ROWS
    mesh = plsc.VectorSubcoreMesh(core_axis_name="c", subcore_axis_name="s")

    @functools.partial(
        pl.kernel, mesh=mesh,
        out_type=jax.ShapeDtypeStruct((B * T,), jnp.float32),
        compiler_params=pltpu.CompilerParams(needs_layout_passes=False),
        scratch_types=[
            pltpu.VMEM((_CHUNK * D,), jnp.float32),
            pltpu.VMEM((D,), jnp.float32),
            pltpu.VMEM((_CHUNK,), jnp.float32),
        ],
    )
    def sc_scores(x_hbm, w_hbm, out_hbm, buf, w_v, stage):
        wid = lax.axis_index("s") * info.num_cores + lax.axis_index("c")
        b = wid // tiles_per_batch
        j = wid % tiles_per_batch
        row0 = b * T + t0 + j * rows_per_tile
        pltpu.sync_copy(w_hbm, w_v)

        iota16 = lax.broadcasted_iota(jnp.int32, (16,), 0)
        rowbase = iota16 * D

        def dc_body(dc, svec):
            wv = w_v[pl.ds(dc * 16, 16)]
            dbase = rowbase + dc * 16
            for i in range(16):
                col = plsc.load_gather(buf, [dbase + i])
                u = lax.bitcast_convert_type(col, jnp.int32)
                u = (u + 0x7FFF + ((u >> 16) & 1)) & jnp.int32(-65536)
                col = lax.bitcast_convert_type(u, jnp.float32)
                svec = svec + col * wv[i]
            return svec

        def chunk_body(g, carry):
            base = row0 + g * _CHUNK
            pltpu.sync_copy(x_hbm.at[pl.ds(base * D, _CHUNK * D)], buf)
            svec = lax.fori_loop(0, D // 16, dc_body,
                                 jnp.zeros((16,), jnp.float32))
            stage[...] = svec
            pltpu.sync_copy(stage, out_hbm.at[pl.ds(base, _CHUNK)])
            return carry

        lax.fori_loop(0, n_chunks, chunk_body, 0)

    return sc_scores


def _fused_body(x_ref, scsc_ref, w_ref, b_ref, s_ref, emph_ref, *, k, t0):
    x = x_ref[0]
    w = w_ref[...]
    bias = b_ref[0]

    tc_scores = jax.lax.dot_general(
        w, x, (((1,), (1,)), ((), ())),
        preferred_element_type=jnp.float32)
    col = lax.broadcasted_iota(jnp.int32, tc_scores.shape, 1)
    scores = jnp.where(col >= t0, scsc_ref[0], tc_scores)
    e = jnp.tanh(scores + bias)

    p = jnp.exp(e)
    z = jnp.sum(p, axis=1, keepdims=True)
    a = p * (1.0 / z)

    ai = jax.lax.bitcast_convert_type(a, jnp.int32)
    j8 = jax.lax.broadcasted_iota(jnp.int32, (8, 1), 0)

    def round3(r, prefix):
        shift = 27 - 3 * r
        cand = prefix | (j8 << shift)
        cnt = jnp.sum((ai >= cand).astype(jnp.int32), axis=1, keepdims=True)
        best = jnp.max(jnp.where(cnt >= k, cand, 0), axis=0, keepdims=True)
        return jnp.broadcast_to(best, (8, 1))

    prefix = jax.lax.fori_loop(0, 10, round3, jnp.zeros((8, 1), jnp.int32),
                               unroll=True)
    kth = prefix[0:1]

    emph = jnp.where(ai >= kth, a * _EMPHASIS, a)
    emph_ref[0] = emph

    s_ref[0] = jax.lax.dot_general(
        emph, x, (((1,), (0,)), ((), ())),
        preferred_element_type=jnp.float32)


@jax.jit
def kernel(x, W, b):
    B, T, D = x.shape
    k = max(int(T * _TOPK_PCT), 1)
    t0 = T - _SC_ROWS
    w_row = W.reshape(1, D)
    w_col = W.reshape(D)

    sc = _sc_scores_kernel_factory(B, T, D)
    sc_scores = sc(x.reshape(B * T * D), w_col).reshape(B, 1, T)

    body = functools.partial(_fused_body, k=k, t0=t0)
    summed, emph = pl.pallas_call(
        body,
        grid=(B,),
        in_specs=[
            pl.BlockSpec((1, T, D), lambda b_: (b_, 0, 0)),
            pl.BlockSpec((1, 1, T), lambda b_: (b_, 0, 0)),
            pl.BlockSpec((1, D), lambda b_: (0, 0)),
            pl.BlockSpec(memory_space=pltpu.SMEM),
        ],
        out_specs=[
            pl.BlockSpec((1, 1, D), lambda b_: (b_, 0, 0)),
            pl.BlockSpec((1, 1, T), lambda b_: (b_, 0, 0)),
        ],
        out_shape=[
            jax.ShapeDtypeStruct((B, 1, D), jnp.float32),
            jax.ShapeDtypeStruct((B, 1, T), jnp.float32),
        ],
        compiler_params=pltpu.CompilerParams(
            dimension_semantics=("arbitrary",),
        ),
    )(x, sc_scores, w_row, b)
    return (summed.reshape(B, D), emph.reshape(B, T))


# speedup vs baseline: 5.6517x; 5.6517x over previous
import functools

import jax
import jax.numpy as jnp
from jax.experimental import pallas as pl
from jax.experimental.pallas import tpu as pltpu

_EMPHASIS = 1.5
_TOPK_PCT = 0.2


def _fused_body(x1_ref, x2_ref, w_ref, b_ref, s_ref, emph_ref, *, k):
    x1 = x1_ref[0]
    x2 = x2_ref[0]
    h = x1.shape[1]
    w1 = w_ref[:, 0:h]
    w2 = w_ref[:, h:2 * h]
    bias = b_ref[0]

    nt = (((1,), (1,)), ((), ()))
    scores = (jax.lax.dot_general(w1, x1, nt, preferred_element_type=jnp.float32)
              + jax.lax.dot_general(w2, x2, nt, preferred_element_type=jnp.float32))
    e = jnp.tanh(scores + bias)

    p = jnp.exp(e)
    z = jnp.sum(p, axis=1, keepdims=True)
    a = p * (1.0 / z)

    ai = jax.lax.bitcast_convert_type(a, jnp.int32)
    j8 = jax.lax.broadcasted_iota(jnp.int32, (8, 1), 0)

    def round3(r, prefix):
        shift = 27 - 3 * r
        cand = prefix | (j8 << shift)
        cnt = jnp.sum((ai >= cand).astype(jnp.int32), axis=1, keepdims=True)
        best = jnp.max(jnp.where(cnt >= k, cand, 0), axis=0, keepdims=True)
        return jnp.broadcast_to(best, (8, 1))

    prefix = jax.lax.fori_loop(0, 10, round3, jnp.zeros((8, 1), jnp.int32),
                               unroll=True)
    kth = prefix[0:1]

    emph = jnp.where(ai >= kth, a * _EMPHASIS, a)
    emph_ref[0] = emph

    nn = (((1,), (0,)), ((), ()))
    s_ref[0, :, 0:h] = jax.lax.dot_general(
        emph, x1, nn, preferred_element_type=jnp.float32)
    s_ref[0, :, h:2 * h] = jax.lax.dot_general(
        emph, x2, nn, preferred_element_type=jnp.float32)


@jax.jit
def kernel(x, W, b):
    B, T, D = x.shape
    h = D // 2
    k = max(int(T * _TOPK_PCT), 1)
    w_row = W.reshape(1, D)
    body = functools.partial(_fused_body, k=k)
    summed, emph = pl.pallas_call(
        body,
        grid=(B,),
        in_specs=[
            pl.BlockSpec((1, T, h), lambda b_: (b_, 0, 0)),
            pl.BlockSpec((1, T, h), lambda b_: (b_, 0, 1)),
            pl.BlockSpec((1, D), lambda b_: (0, 0)),
            pl.BlockSpec(memory_space=pltpu.SMEM),
        ],
        out_specs=[
            pl.BlockSpec((1, 1, D), lambda b_: (b_, 0, 0)),
            pl.BlockSpec((1, 1, T), lambda b_: (b_, 0, 0)),
        ],
        out_shape=[
            jax.ShapeDtypeStruct((B, 1, D), jnp.float32),
            jax.ShapeDtypeStruct((B, 1, T), jnp.float32),
        ],
        compiler_params=pltpu.CompilerParams(
            dimension_semantics=("arbitrary",),
        ),
    )(x, x, w_row, b)
    return (summed.reshape(B, D), emph.reshape(B, T))


# speedup vs baseline: 5.6539x; 1.0004x over previous
import functools

import jax
import jax.numpy as jnp
from jax.experimental import pallas as pl
from jax.experimental.pallas import tpu as pltpu

_EMPHASIS = 1.5
_TOPK_PCT = 0.2


def _fused_body(x1_ref, x2_ref, w_ref, b_ref, s_ref, emph_ref, *, k):
    x1 = x1_ref[0]
    x2 = x2_ref[0]
    h = x1.shape[1]
    w1 = w_ref[:, 0:h]
    w2 = w_ref[:, h:2 * h]
    bias = b_ref[0]

    nt = (((1,), (1,)), ((), ()))
    scores = (jax.lax.dot_general(w1, x1, nt, preferred_element_type=jnp.float32)
              + jax.lax.dot_general(w2, x2, nt, preferred_element_type=jnp.float32))
    e = jnp.tanh(scores + bias)

    p = jnp.exp(e)
    z = jnp.sum(p, axis=1, keepdims=True)
    a = p * (1.0 / z)

    ai = jax.lax.bitcast_convert_type(a, jnp.int32)
    j8 = jax.lax.broadcasted_iota(jnp.int32, (8, 1), 0)

    def round3(r, prefix):
        shift = 27 - 3 * r
        cand = prefix | (j8 << shift)
        cnt = jnp.sum((ai >= cand).astype(jnp.int32), axis=1, keepdims=True)
        best = jnp.max(jnp.where(cnt >= k, cand, 0), axis=0, keepdims=True)
        return jnp.broadcast_to(best, (8, 1))

    prefix = jax.lax.fori_loop(0, 10, round3, jnp.zeros((8, 1), jnp.int32),
                               unroll=True)
    kth = prefix[0:1]

    emph = jnp.where(ai >= kth, a * _EMPHASIS, a)
    emph_ref[0] = emph

    nn = (((1,), (0,)), ((), ()))
    s_ref[0, :, 0:h] = jax.lax.dot_general(
        emph, x1, nn, preferred_element_type=jnp.float32)
    s_ref[0, :, h:2 * h] = jax.lax.dot_general(
        emph, x2, nn, preferred_element_type=jnp.float32)


@jax.jit
def kernel(x, W, b):
    B, T, D = x.shape
    h = D // 2
    k = max(int(T * _TOPK_PCT), 1)
    w_row = W.reshape(1, D)
    body = functools.partial(_fused_body, k=k)
    summed, emph = pl.pallas_call(
        body,
        grid=(B,),
        in_specs=[
            pl.BlockSpec((1, T, h), lambda b_: (b_, 0, 0)),
            pl.BlockSpec((1, T, h), lambda b_: (b_, 0, 1)),
            pl.BlockSpec((1, D), lambda b_: (0, 0)),
            pl.BlockSpec(memory_space=pltpu.SMEM),
        ],
        out_specs=[
            pl.BlockSpec((1, 1, D), lambda b_: (b_, 0, 0)),
            pl.BlockSpec((1, 1, T), lambda b_: (b_, 0, 0)),
        ],
        out_shape=[
            jax.ShapeDtypeStruct((B, 1, D), jnp.float32),
            jax.ShapeDtypeStruct((B, 1, T), jnp.float32),
        ],
        compiler_params=pltpu.CompilerParams(
            dimension_semantics=("parallel",),
        ),
    )(x, x, w_row, b)
    return (summed.reshape(B, D), emph.reshape(B, T))
